# baseline (speedup 1.0000x reference)
import functools

import jax
import jax.numpy as jnp
from jax import lax
from jax.experimental import pallas as pl
from jax.experimental.pallas import tpu as pltpu
from jax.experimental.pallas import tpu_sc as plsc

B = 16384
V = 100000
D = 16
F = 26
DENSE = 13
OUT_W = F * D + DENSE

NC = 2
NS = 16
L = 16
NW = NC * NS
ROWS_PER_W = B // NW
C = 128
NCHUNK = ROWS_PER_W // C

_mesh = plsc.VectorSubcoreMesh(core_axis_name="c", subcore_axis_name="s")


@functools.partial(
    pl.kernel,
    out_type=jax.ShapeDtypeStruct((B, OUT_W), jnp.float32),
    mesh=_mesh,
    scratch_types=[
        pltpu.VMEM((F, C), jnp.int32),
        pltpu.VMEM((F, C, D), jnp.float32),
        pltpu.VMEM((C, DENSE), jnp.float32),
        pltpu.SemaphoreType.DMA,
    ],
    compiler_params=pltpu.CompilerParams(use_tc_tiling_on_sc=False),
)
def _emb_concat(idx_hbm, dense_hbm, tbl_hbm, out_hbm, idx_v, rows_v, dense_v, sem):
    wid = lax.axis_index("s") * NC + lax.axis_index("c")

    def chunk_body(ci, carry):
        base = wid * ROWS_PER_W + ci * C
        pltpu.sync_copy(idx_hbm.at[:, pl.ds(base, C)], idx_v)

        def field_body(f, carry2):
            off = f * V
            for j in range(C // L):
                sl = pl.ds(j * L, L)
                idx_v[f, sl] = idx_v[f, sl] + off
            pltpu.async_copy(tbl_hbm.at[idx_v.at[f]], rows_v.at[f], sem).wait()
            pltpu.sync_copy(rows_v.at[f], out_hbm.at[pl.ds(base, C), pl.ds(f * D, D)])
            return carry2

        lax.fori_loop(0, F, field_body, 0, unroll=False)

        pltpu.sync_copy(dense_hbm.at[pl.ds(base, C)], dense_v)
        pltpu.sync_copy(dense_v, out_hbm.at[pl.ds(base, C), pl.ds(F * D, DENSE)])
        return carry

    lax.fori_loop(0, NCHUNK, chunk_body, 0, unroll=False)


def kernel(sparse_fields, dense_0, tables):
    idx = sparse_fields.astype(jnp.int32)
    tbl = tables.reshape(F * V, D)
    return _emb_concat(idx, dense_0, tbl)


# speedup vs baseline: 1.0760x; 1.0760x over previous
import functools

import jax
import jax.numpy as jnp
from jax import lax
from jax.experimental import pallas as pl
from jax.experimental.pallas import tpu as pltpu
from jax.experimental.pallas import tpu_sc as plsc

B = 16384
V = 100000
D = 16
F = 26
DENSE = 13
OUT_W = F * D + DENSE

NC = 2
NS = 16
L = 16
NW = NC * NS
ROWS_PER_W = B // NW
C = 128
NCHUNK = ROWS_PER_W // C

_mesh = plsc.VectorSubcoreMesh(core_axis_name="c", subcore_axis_name="s")


@functools.partial(
    pl.kernel,
    out_type=jax.ShapeDtypeStruct((B, OUT_W), jnp.float32),
    mesh=_mesh,
    scratch_types=[
        pltpu.VMEM((F, ROWS_PER_W), jnp.int32),
        pltpu.VMEM((F, C, D), jnp.float32),
        pltpu.VMEM((C, DENSE), jnp.float32),
        pltpu.SemaphoreType.DMA,
        pltpu.SemaphoreType.DMA,
    ],
    compiler_params=pltpu.CompilerParams(use_tc_tiling_on_sc=False),
)
def _emb_concat(idx_hbm, dense_hbm, tbl_hbm, out_hbm, idx_v, rows_v, dense_v,
                gsem, wsem):
    wid = lax.axis_index("s") * NC + lax.axis_index("c")
    row0 = wid * ROWS_PER_W

    pltpu.sync_copy(idx_hbm.at[:, pl.ds(row0, ROWS_PER_W)], idx_v)

    def off_body(f, carry):
        off = f * V
        for j in range(ROWS_PER_W // L):
            sl = pl.ds(j * L, L)
            idx_v[f, sl] = idx_v[f, sl] + off
        return carry

    lax.fori_loop(0, F, off_body, 0, unroll=False)

    def chunk_body(ci, carry):
        base = row0 + ci * C

        def fire_body(f, carry2):
            pltpu.make_async_copy(
                tbl_hbm.at[idx_v.at[f, pl.ds(ci * C, C)]],
                rows_v.at[f], gsem).start()
            return carry2

        def drain_write_body(f, carry2):
            pltpu.make_async_copy(
                tbl_hbm.at[idx_v.at[f, pl.ds(ci * C, C)]],
                rows_v.at[f], gsem).wait()
            pltpu.make_async_copy(
                rows_v.at[f],
                out_hbm.at[pl.ds(base, C), pl.ds(f * D, D)], wsem).start()
            return carry2

        def wdrain_body(f, carry2):
            pltpu.make_async_copy(
                rows_v.at[f],
                out_hbm.at[pl.ds(base, C), pl.ds(f * D, D)], wsem).wait()
            return carry2

        lax.fori_loop(0, F, fire_body, 0, unroll=False)
        lax.fori_loop(0, F, drain_write_body, 0, unroll=False)

        pltpu.sync_copy(dense_hbm.at[pl.ds(base, C)], dense_v)
        pltpu.sync_copy(dense_v, out_hbm.at[pl.ds(base, C), pl.ds(F * D, DENSE)])

        lax.fori_loop(0, F, wdrain_body, 0, unroll=False)
        return carry

    lax.fori_loop(0, NCHUNK, chunk_body, 0, unroll=False)


def kernel(sparse_fields, dense_0, tables):
    idx = sparse_fields.astype(jnp.int32)
    tbl = tables.reshape(F * V, D)
    return _emb_concat(idx, dense_0, tbl)
